# baseline (speedup 1.0000x reference)
import functools

import jax
import jax.numpy as jnp
from jax import lax
from jax.experimental import pallas as pl
from jax.experimental.pallas import tpu as pltpu
from jax.experimental.pallas import tpu_sc as plsc

B, Q, C, R, V = 64, 300, 92, 100, 117
NC, NS, L = 2, 16, 16
NW = NC * NS
BPW = B // NW
NG = (R + L - 1) // L
NCLS = 81

_mesh = plsc.VectorSubcoreMesh(
    core_axis_name="c", subcore_axis_name="s", num_cores=NC, num_subcores=NS)


def _splat(x, dtype=jnp.int32):
    return jnp.zeros((L,), dtype) + x


def _iota():
    return lax.iota(jnp.int32, L)


def _batch_scratch():
    return [
        pltpu.VMEM((R, 2), jnp.int32),
        pltpu.VMEM((Q, C), jnp.float32),
        pltpu.VMEM((Q, 4), jnp.float32),
        pltpu.VMEM((R, V), jnp.float32),
        pltpu.VMEM((R,), jnp.float32),
        pltpu.VMEM((2 * R,), jnp.int32),
        pltpu.VMEM((2 * R, 4), jnp.float32),
        pltpu.VMEM((R, V), jnp.float32),
    ]


@functools.partial(
    pl.kernel,
    out_type=(
        jax.ShapeDtypeStruct((B, 2 * R), jnp.int32),
        jax.ShapeDtypeStruct((B, 2 * R, 4), jnp.float32),
        jax.ShapeDtypeStruct((B, R, V), jnp.float32),
    ),
    mesh=_mesh,
    compiler_params=pltpu.CompilerParams(
        needs_layout_passes=False, use_tc_tiling_on_sc=False),
    scratch_types=[pltpu.VMEM((B, 2), jnp.float32)] + _batch_scratch()
    + _batch_scratch() + [pltpu.SemaphoreType.DMA],
)
def _postprocess(logits_hbm, boxes_hbm, actions_hbm, pairs_hbm, ts_hbm,
                 l_hbm, b_hbm, vs_hbm, ts_v, *rest):
    per_batch = (rest[0:8], rest[8:16])
    sem = rest[16]
    wid = lax.axis_index("s") * NC + lax.axis_index("c")
    b0 = wid * BPW

    copies = []
    for j in range(BPW):
        pairs_v, logits_v, boxes_v, act_v = per_batch[j][:4]
        copies.append(pltpu.async_copy(logits_hbm.at[b0 + j], logits_v, sem))
        copies.append(pltpu.async_copy(actions_hbm.at[b0 + j], act_v, sem))
        copies.append(pltpu.async_copy(boxes_hbm.at[b0 + j], boxes_v, sem))
        copies.append(pltpu.async_copy(pairs_hbm.at[b0 + j], pairs_v, sem))
    pltpu.sync_copy(ts_hbm, ts_v)
    for c in copies:
        c.wait()

    for j in range(BPW):
        pairs_v, logits_v, boxes_v, act_v, scores_v, l_buf, b_buf, vs_buf = (
            per_batch[j])
        img_h = plsc.load_gather(ts_v, [_splat(b0 + j), _splat(0)])
        img_w = plsc.load_gather(ts_v, [_splat(b0 + j), _splat(1)])

        for g in range(NG):
            pi_raw = _iota() + g * L
            mask = (pi_raw < R) if g == NG - 1 else None
            pi = jnp.minimum(pi_raw, R - 1)

            plsc.store_scatter(l_buf, [pi], _splat(0), mask=mask)
            for side in range(2):
                bi = plsc.load_gather(pairs_v, [pi, _splat(side)])
                cx = plsc.load_gather(boxes_v, [bi, _splat(0)])
                cy = plsc.load_gather(boxes_v, [bi, _splat(1)])
                w = plsc.load_gather(boxes_v, [bi, _splat(2)])
                h = plsc.load_gather(boxes_v, [bi, _splat(3)])
                orow = pi + side * R
                plsc.store_scatter(b_buf, [orow, _splat(0)],
                                   (cx - 0.5 * w) * img_w, mask=mask)
                plsc.store_scatter(b_buf, [orow, _splat(1)],
                                   (cy - 0.5 * h) * img_h, mask=mask)
                plsc.store_scatter(b_buf, [orow, _splat(2)],
                                   (cx + 0.5 * w) * img_w, mask=mask)
                plsc.store_scatter(b_buf, [orow, _splat(3)],
                                   (cy + 0.5 * h) * img_h, mask=mask)

            oi = plsc.load_gather(pairs_v, [pi, _splat(1)])

            def maxbody(c, carry, oi=oi):
                m, am = carry
                v = plsc.load_gather(logits_v, [oi, _splat(c)])
                gt = v > m
                return jnp.where(gt, v, m), jnp.where(gt, _splat(c), am)

            m80, am = lax.fori_loop(
                0, NCLS - 1, maxbody,
                (_splat(-jnp.inf, jnp.float32), _splat(0)))
            v80 = plsc.load_gather(logits_v, [oi, _splat(NCLS - 1)])
            mall = jnp.maximum(m80, v80)

            def sumbody(c, s, oi=oi, mall=mall):
                v = plsc.load_gather(logits_v, [oi, _splat(c)])
                return s + jnp.exp(v - mall)

            s = lax.fori_loop(0, NCLS, sumbody, _splat(0.0, jnp.float32))
            score = jnp.exp(m80 - mall) / s
            plsc.store_scatter(scores_v, [pi], score, mask=mask)
            plsc.store_scatter(l_buf, [pi + R], am, mask=mask)

        offs = [0, 16, 32, 48, 64, 80, 96, V - L]

        def vsbody(p, _, act_v=act_v, vs_buf=vs_buf, scores_v=scores_v):
            sc = plsc.load_gather(scores_v, [_splat(p)])
            for off in offs:
                ci = _iota() + off
                v = plsc.load_gather(act_v, [_splat(p), ci])
                out = sc / (1.0 + jnp.exp(-v))
                plsc.store_scatter(vs_buf, [_splat(p), ci], out)
            return 0

        lax.fori_loop(0, R, vsbody, 0)

        pltpu.sync_copy(l_buf, l_hbm.at[b0 + j])
        pltpu.sync_copy(b_buf, b_hbm.at[b0 + j])
        pltpu.sync_copy(vs_buf, vs_hbm.at[b0 + j])


def kernel(pred_logits, pred_boxes, pred_actions, pred_rel_pairs, target_sizes):
    return _postprocess(pred_logits, pred_boxes, pred_actions,
                        pred_rel_pairs, target_sizes)


# speedup vs baseline: 1.0545x; 1.0545x over previous
import functools

import jax
import jax.numpy as jnp
from jax import lax
from jax.experimental import pallas as pl
from jax.experimental.pallas import tpu as pltpu
from jax.experimental.pallas import tpu_sc as plsc

B, Q, C, R, V = 64, 300, 92, 100, 117
NC, NS, L = 2, 16, 16
NW = NC * NS
BPW = B // NW
NG = (R + L - 1) // L
NCLS = 81

_mesh = plsc.VectorSubcoreMesh(
    core_axis_name="c", subcore_axis_name="s", num_cores=NC, num_subcores=NS)


def _splat(x, dtype=jnp.int32):
    return jnp.zeros((L,), dtype) + x


def _iota():
    return lax.iota(jnp.int32, L)


def _batch_scratch():
    return [
        pltpu.VMEM((R, 2), jnp.int32),
        pltpu.VMEM((Q, C), jnp.float32),
        pltpu.VMEM((Q, 4), jnp.float32),
        pltpu.VMEM((R, V), jnp.float32),
        pltpu.VMEM((R,), jnp.float32),
        pltpu.VMEM((2 * R,), jnp.int32),
        pltpu.VMEM((2 * R, 4), jnp.float32),
        pltpu.VMEM((R, V), jnp.float32),
    ]


@functools.partial(
    pl.kernel,
    out_type=(
        jax.ShapeDtypeStruct((B, 2 * R), jnp.int32),
        jax.ShapeDtypeStruct((B, 2 * R, 4), jnp.float32),
        jax.ShapeDtypeStruct((B, R, V), jnp.float32),
    ),
    mesh=_mesh,
    compiler_params=pltpu.CompilerParams(
        needs_layout_passes=False, use_tc_tiling_on_sc=False),
    scratch_types=[pltpu.VMEM((B, 2), jnp.float32)] + _batch_scratch()
    + _batch_scratch()
    + [pltpu.SemaphoreType.DMA, pltpu.SemaphoreType.DMA,
       pltpu.SemaphoreType.DMA],
)
def _postprocess(logits_hbm, boxes_hbm, actions_hbm, pairs_hbm, ts_hbm,
                 l_hbm, b_hbm, vs_hbm, ts_v, *rest):
    per_batch = (rest[0:8], rest[8:16])
    sems = rest[16:18]
    sem_out = rest[18]
    wid = lax.axis_index("s") * NC + lax.axis_index("c")
    b0 = wid * BPW

    copies = []
    for j in range(BPW):
        pairs_v, logits_v, boxes_v, act_v = per_batch[j][:4]
        copies.append([
            pltpu.async_copy(logits_hbm.at[b0 + j], logits_v, sems[j]),
            pltpu.async_copy(actions_hbm.at[b0 + j], act_v, sems[j]),
            pltpu.async_copy(boxes_hbm.at[b0 + j], boxes_v, sems[j]),
            pltpu.async_copy(pairs_hbm.at[b0 + j], pairs_v, sems[j]),
        ])
    pltpu.sync_copy(ts_hbm, ts_v)
    out_copies = []

    for j in range(BPW):
        pairs_v, logits_v, boxes_v, act_v, scores_v, l_buf, b_buf, vs_buf = (
            per_batch[j])
        for c in copies[j]:
            c.wait()
        img_h = plsc.load_gather(ts_v, [_splat(b0 + j), _splat(0)])
        img_w = plsc.load_gather(ts_v, [_splat(b0 + j), _splat(1)])

        pis, masks, ois = [], [], []
        for g in range(NG):
            pi_raw = _iota() + g * L
            masks.append((pi_raw < R) if g == NG - 1 else None)
            pis.append(jnp.minimum(pi_raw, R - 1))
            ois.append(plsc.load_gather(pairs_v, [pis[g], _splat(1)]))

        for g in range(NG):
            pi, mask = pis[g], masks[g]
            plsc.store_scatter(l_buf, [pi], _splat(0), mask=mask)
            for side in range(2):
                bi = (plsc.load_gather(pairs_v, [pi, _splat(0)])
                      if side == 0 else ois[g])
                cx = plsc.load_gather(boxes_v, [bi, _splat(0)])
                cy = plsc.load_gather(boxes_v, [bi, _splat(1)])
                w = plsc.load_gather(boxes_v, [bi, _splat(2)])
                h = plsc.load_gather(boxes_v, [bi, _splat(3)])
                orow = pi + side * R
                plsc.store_scatter(b_buf, [orow, _splat(0)],
                                   (cx - 0.5 * w) * img_w, mask=mask)
                plsc.store_scatter(b_buf, [orow, _splat(1)],
                                   (cy - 0.5 * h) * img_h, mask=mask)
                plsc.store_scatter(b_buf, [orow, _splat(2)],
                                   (cx + 0.5 * w) * img_w, mask=mask)
                plsc.store_scatter(b_buf, [orow, _splat(3)],
                                   (cy + 0.5 * h) * img_h, mask=mask)

        U1 = 4

        def maxbody(i, carry, ois=ois):
            ms, ams = carry
            ms, ams = list(ms), list(ams)
            for u in range(U1):
                cs = _splat(i * U1 + u)
                for g in range(NG):
                    v = plsc.load_gather(logits_v, [ois[g], cs])
                    gt = v > ms[g]
                    ms[g] = jnp.where(gt, v, ms[g])
                    ams[g] = jnp.where(gt, cs, ams[g])
            return tuple(ms), tuple(ams)

        m80s, ams = lax.fori_loop(
            0, (NCLS - 1) // U1, maxbody,
            (tuple(_splat(-jnp.inf, jnp.float32) for _ in range(NG)),
             tuple(_splat(0) for _ in range(NG))))
        malls = []
        for g in range(NG):
            v80 = plsc.load_gather(logits_v, [ois[g], _splat(NCLS - 1)])
            malls.append(jnp.maximum(m80s[g], v80))
        malls = tuple(malls)

        U2 = 3

        def sumbody(i, ss, ois=ois, malls=malls):
            ss = list(ss)
            for u in range(U2):
                cs = _splat(i * U2 + u)
                for g in range(NG):
                    v = plsc.load_gather(logits_v, [ois[g], cs])
                    ss[g] = ss[g] + jnp.exp(v - malls[g])
            return tuple(ss)

        ss = lax.fori_loop(
            0, NCLS // U2, sumbody,
            tuple(_splat(0.0, jnp.float32) for _ in range(NG)))
        for g in range(NG):
            score = jnp.exp(m80s[g] - malls[g]) / ss[g]
            plsc.store_scatter(scores_v, [pis[g]], score, mask=masks[g])
            plsc.store_scatter(l_buf, [pis[g] + R], ams[g], mask=masks[g])

        offs = [0, 16, 32, 48, 64, 80, 96, V - L]
        UP = 2

        def vsbody(p0, _, act_v=act_v, vs_buf=vs_buf, scores_v=scores_v):
            for u in range(UP):
                p = p0 * UP + u
                sc = plsc.load_gather(scores_v, [_splat(p)])
                for off in offs:
                    ci = _iota() + off
                    v = plsc.load_gather(act_v, [_splat(p), ci])
                    out = sc / (1.0 + jnp.exp(-v))
                    plsc.store_scatter(vs_buf, [_splat(p), ci], out)
            return 0

        lax.fori_loop(0, R // UP, vsbody, 0)

        out_copies.extend([
            pltpu.async_copy(l_buf, l_hbm.at[b0 + j], sem_out),
            pltpu.async_copy(b_buf, b_hbm.at[b0 + j], sem_out),
            pltpu.async_copy(vs_buf, vs_hbm.at[b0 + j], sem_out),
        ])

    for c in out_copies:
        c.wait()


def kernel(pred_logits, pred_boxes, pred_actions, pred_rel_pairs, target_sizes):
    return _postprocess(pred_logits, pred_boxes, pred_actions,
                        pred_rel_pairs, target_sizes)


# speedup vs baseline: 1.2780x; 1.2119x over previous
import functools

import jax
import jax.numpy as jnp
from jax import lax
from jax.experimental import pallas as pl
from jax.experimental.pallas import tpu as pltpu
from jax.experimental.pallas import tpu_sc as plsc

B, Q, C, R, V = 64, 300, 92, 100, 117
NC, NS, L = 2, 16, 16
NW = NC * NS
BPW = B // NW
NG = (R + L - 1) // L
NCLS = 81

_mesh = plsc.VectorSubcoreMesh(
    core_axis_name="c", subcore_axis_name="s", num_cores=NC, num_subcores=NS)


def _splat(x, dtype=jnp.int32):
    return jnp.zeros((L,), dtype) + x


def _iota():
    return lax.iota(jnp.int32, L)


def _batch_scratch():
    return [
        pltpu.VMEM((R, 2), jnp.int32),
        pltpu.VMEM((Q, C), jnp.float32),
        pltpu.VMEM((Q, 4), jnp.float32),
        pltpu.VMEM((R, V), jnp.float32),
        pltpu.VMEM((R,), jnp.float32),
        pltpu.VMEM((2 * R,), jnp.int32),
        pltpu.VMEM((2 * R, 4), jnp.float32),
        pltpu.VMEM((R, V), jnp.float32),
    ]


@functools.partial(
    pl.kernel,
    out_type=(
        jax.ShapeDtypeStruct((B, 2 * R), jnp.int32),
        jax.ShapeDtypeStruct((B, 2 * R, 4), jnp.float32),
        jax.ShapeDtypeStruct((B, R, V), jnp.float32),
    ),
    mesh=_mesh,
    compiler_params=pltpu.CompilerParams(
        needs_layout_passes=False, use_tc_tiling_on_sc=False),
    scratch_types=[pltpu.VMEM((B, 2), jnp.float32)] + _batch_scratch()
    + _batch_scratch()
    + [pltpu.SemaphoreType.DMA, pltpu.SemaphoreType.DMA,
       pltpu.SemaphoreType.DMA],
)
def _postprocess(logits_hbm, boxes_hbm, actions_hbm, pairs_hbm, ts_hbm,
                 l_hbm, b_hbm, vs_hbm, ts_v, *rest):
    per_batch = (rest[0:8], rest[8:16])
    sems = rest[16:18]
    sem_out = rest[18]
    wid = lax.axis_index("s") * NC + lax.axis_index("c")
    b0 = wid * BPW

    copies = []
    for j in range(BPW):
        pairs_v, logits_v, boxes_v, act_v = per_batch[j][:4]
        copies.append([
            pltpu.async_copy(logits_hbm.at[b0 + j], logits_v, sems[j]),
            pltpu.async_copy(actions_hbm.at[b0 + j], act_v, sems[j]),
            pltpu.async_copy(boxes_hbm.at[b0 + j], boxes_v, sems[j]),
            pltpu.async_copy(pairs_hbm.at[b0 + j], pairs_v, sems[j]),
        ])
    pltpu.sync_copy(ts_hbm, ts_v)
    out_copies = []

    for j in range(BPW):
        pairs_v, logits_v, boxes_v, act_v, scores_v, l_buf, b_buf, vs_buf = (
            per_batch[j])
        for c in copies[j]:
            c.wait()
        img_h = plsc.load_gather(ts_v, [_splat(b0 + j), _splat(0)])
        img_w = plsc.load_gather(ts_v, [_splat(b0 + j), _splat(1)])

        pis, masks, ois = [], [], []
        for g in range(NG):
            pi_raw = _iota() + g * L
            masks.append((pi_raw < R) if g == NG - 1 else None)
            pis.append(jnp.minimum(pi_raw, R - 1))
            ois.append(plsc.load_gather(pairs_v, [pis[g], _splat(1)]))

        @plsc.parallel_loop(0, NG)
        def _boxes(g, pairs_v=pairs_v, boxes_v=boxes_v, l_buf=l_buf,
                   b_buf=b_buf, img_h=img_h, img_w=img_w):
            pi_raw = _iota() + g * L
            mask = pi_raw < R
            pi = jnp.minimum(pi_raw, R - 1)
            plsc.store_scatter(l_buf, [pi], _splat(0), mask=mask)
            for side in range(2):
                bi = plsc.load_gather(pairs_v, [pi, _splat(side)])
                cx = plsc.load_gather(boxes_v, [bi, _splat(0)])
                cy = plsc.load_gather(boxes_v, [bi, _splat(1)])
                w = plsc.load_gather(boxes_v, [bi, _splat(2)])
                h = plsc.load_gather(boxes_v, [bi, _splat(3)])
                orow = pi + side * R
                plsc.store_scatter(b_buf, [orow, _splat(0)],
                                   (cx - 0.5 * w) * img_w, mask=mask)
                plsc.store_scatter(b_buf, [orow, _splat(1)],
                                   (cy - 0.5 * h) * img_h, mask=mask)
                plsc.store_scatter(b_buf, [orow, _splat(2)],
                                   (cx + 0.5 * w) * img_w, mask=mask)
                plsc.store_scatter(b_buf, [orow, _splat(3)],
                                   (cy + 0.5 * h) * img_h, mask=mask)

        init = (tuple(_splat(-jnp.inf, jnp.float32) for _ in range(NG)),
                tuple(_splat(0) for _ in range(NG)))

        @plsc.parallel_loop(0, NCLS - 1, unroll=4, carry=init)
        def maxstate(c, carry, ois=ois, logits_v=logits_v):
            ms, ams = carry
            ms, ams = list(ms), list(ams)
            cs = _splat(c)
            for g in range(NG):
                v = plsc.load_gather(logits_v, [ois[g], cs])
                gt = v > ms[g]
                ms[g] = jnp.where(gt, v, ms[g])
                ams[g] = jnp.where(gt, cs, ams[g])
            return tuple(ms), tuple(ams)

        m80s, ams = maxstate
        malls = []
        for g in range(NG):
            v80 = plsc.load_gather(logits_v, [ois[g], _splat(NCLS - 1)])
            malls.append(jnp.maximum(m80s[g], v80))
        malls = tuple(malls)

        @plsc.parallel_loop(0, NCLS, unroll=3,
                            carry=tuple(_splat(0.0, jnp.float32)
                                        for _ in range(NG)))
        def ss(c, ss_c, ois=ois, malls=malls, logits_v=logits_v):
            ss_c = list(ss_c)
            cs = _splat(c)
            for g in range(NG):
                v = plsc.load_gather(logits_v, [ois[g], cs])
                ss_c[g] = ss_c[g] + jnp.exp(v - malls[g])
            return tuple(ss_c)

        for g in range(NG):
            score = jnp.exp(m80s[g] - malls[g]) / ss[g]
            plsc.store_scatter(scores_v, [pis[g]], score, mask=masks[g])
            plsc.store_scatter(l_buf, [pis[g] + R], ams[g], mask=masks[g])

        offs = [0, 16, 32, 48, 64, 80, 96, V - L]

        @plsc.parallel_loop(0, R, unroll=2)
        def _vs(p, act_v=act_v, vs_buf=vs_buf, scores_v=scores_v):
            sc = plsc.load_gather(scores_v, [_splat(p)])
            for off in offs:
                ci = _iota() + off
                v = plsc.load_gather(act_v, [_splat(p), ci])
                out = sc / (1.0 + jnp.exp(-v))
                plsc.store_scatter(vs_buf, [_splat(p), ci], out)

        out_copies.extend([
            pltpu.async_copy(l_buf, l_hbm.at[b0 + j], sem_out),
            pltpu.async_copy(b_buf, b_hbm.at[b0 + j], sem_out),
            pltpu.async_copy(vs_buf, vs_hbm.at[b0 + j], sem_out),
        ])

    for c in out_copies:
        c.wait()


def kernel(pred_logits, pred_boxes, pred_actions, pred_rel_pairs, target_sizes):
    return _postprocess(pred_logits, pred_boxes, pred_actions,
                        pred_rel_pairs, target_sizes)
